# baseline (speedup 1.0000x reference)
import functools

import jax
import jax.numpy as jnp
from jax.experimental import pallas as pl
from jax.experimental.pallas import tpu as pltpu


def _enhancer_kernel(wa_ref, wb_ref, ba_ref, bb_ref, x_ref, o_ref, *, T, BB, C):
    t = jax.lax.broadcasted_iota(jnp.int32, (1, T), 1)
    m_first = t == 0
    m_last = t == T - 1
    zero = jnp.bfloat16(0)
    ba = ba_ref[...]
    bb = bb_ref[...]
    for i in range(BB):
        xa = x_ref[i, C:3 * C].astype(jnp.bfloat16)
        xb = jnp.concatenate(
            [x_ref[i, :C], x_ref[i, 3 * C:]], axis=0).astype(jnp.bfloat16)
        ya = None
        yb = None
        for x2, wref, prev_y in ((xa, wa_ref, "a"), (xb, wb_ref, "b")):
            x_prev = jnp.where(m_first, zero, pltpu.roll(x2, shift=1, axis=1))
            x_next = jnp.where(m_last, zero, pltpu.roll(x2, shift=T - 1, axis=1))
            y = jnp.dot(wref[0], x_prev, preferred_element_type=jnp.float32)
            y = y + jnp.dot(wref[1], x2, preferred_element_type=jnp.float32)
            y = y + jnp.dot(wref[2], x_next, preferred_element_type=jnp.float32)
            if prev_y == "a":
                ya = y
            else:
                yb = y
        o_ref[i, C:3 * C] = ya + ba
        o_ref[i, :C] = yb[:C] + bb[:C]
        o_ref[i, 3 * C:] = yb[C:] + bb[C:]


def kernel(mel_spec, w_taps, bias_col):
    B, M, T = mel_spec.shape
    BB = 8
    C = M // 4

    w_folded = w_taps.at[1].add(jnp.float32(0.7) * jnp.eye(M, dtype=w_taps.dtype))
    wa = w_folded[:, C:3 * C, C:3 * C].astype(jnp.bfloat16)
    wb = jnp.zeros((3, 2 * C, 2 * C), jnp.float32)
    wb = wb.at[:, :C, :C].set(w_folded[:, :C, :C])
    wb = wb.at[:, C:, C:].set(w_folded[:, 3 * C:, 3 * C:])
    wb = wb.astype(jnp.bfloat16)
    ba = bias_col[C:3 * C]
    bb = jnp.concatenate([bias_col[:C], bias_col[3 * C:]], axis=0)

    return pl.pallas_call(
        functools.partial(_enhancer_kernel, T=T, BB=BB, C=C),
        out_shape=jax.ShapeDtypeStruct((B, M, T), mel_spec.dtype),
        grid=(B // BB,),
        in_specs=[
            pl.BlockSpec((3, 2 * C, 2 * C), lambda b: (0, 0, 0)),
            pl.BlockSpec((3, 2 * C, 2 * C), lambda b: (0, 0, 0)),
            pl.BlockSpec((2 * C, 1), lambda b: (0, 0)),
            pl.BlockSpec((2 * C, 1), lambda b: (0, 0)),
            pl.BlockSpec((BB, M, T), lambda b: (b, 0, 0)),
        ],
        out_specs=pl.BlockSpec((BB, M, T), lambda b: (b, 0, 0)),
        compiler_params=pltpu.CompilerParams(
            dimension_semantics=("parallel",),
            vmem_limit_bytes=64 << 20,
        ),
    )(wa, wb, ba, bb, mel_spec)


# speedup vs baseline: 1.0365x; 1.0365x over previous
import functools

import jax
import jax.numpy as jnp
from jax.experimental import pallas as pl
from jax.experimental.pallas import tpu as pltpu


def _enhancer_kernel(w_ref, b_ref, x_ref, o_ref, *, T, BB):
    t = jax.lax.broadcasted_iota(jnp.int32, (1, T), 1)
    m_first = t == 0
    m_last = t == T - 1
    w0, w1, w2 = w_ref[0], w_ref[1], w_ref[2]
    bias = b_ref[...]
    zero = jnp.bfloat16(0)
    for i in range(BB):
        xb = x_ref[i].astype(jnp.bfloat16)
        x_prev = jnp.where(m_first, zero, pltpu.roll(xb, shift=1, axis=1))
        x_next = jnp.where(m_last, zero, pltpu.roll(xb, shift=T - 1, axis=1))
        y = jnp.dot(w0, x_prev, preferred_element_type=jnp.float32)
        y = y + jnp.dot(w1, xb, preferred_element_type=jnp.float32)
        y = y + jnp.dot(w2, x_next, preferred_element_type=jnp.float32)
        o_ref[i] = y + bias


def kernel(mel_spec, w_taps, bias_col):
    B, M, T = mel_spec.shape
    BB = 4
    w_folded = w_taps.at[1].add(jnp.float32(0.7) * jnp.eye(M, dtype=w_taps.dtype))
    w_bf16 = w_folded.astype(jnp.bfloat16)

    return pl.pallas_call(
        functools.partial(_enhancer_kernel, T=T, BB=BB),
        out_shape=jax.ShapeDtypeStruct((B, M, T), mel_spec.dtype),
        grid=(B // BB,),
        in_specs=[
            pl.BlockSpec((3, M, M), lambda b: (0, 0, 0)),
            pl.BlockSpec((M, 1), lambda b: (0, 0)),
            pl.BlockSpec((BB, M, T), lambda b: (b, 0, 0)),
        ],
        out_specs=pl.BlockSpec((BB, M, T), lambda b: (b, 0, 0)),
        compiler_params=pltpu.CompilerParams(
            dimension_semantics=("parallel",),
            vmem_limit_bytes=64 << 20,
        ),
    )(w_bf16, bias_col, mel_spec)


# speedup vs baseline: 1.0971x; 1.0585x over previous
import functools

import jax
import jax.numpy as jnp
from jax.experimental import pallas as pl
from jax.experimental.pallas import tpu as pltpu


def _enhancer_kernel(w_ref, b_ref, x_ref, o_ref, *, T, BB):
    L = BB * T
    t = jax.lax.broadcasted_iota(jnp.int32, (1, L), 1)
    tm = jax.lax.rem(t, T)
    m_first = tm == 0
    m_last = tm == T - 1
    zero = jnp.bfloat16(0)

    xs = jnp.concatenate(
        [x_ref[i].astype(jnp.bfloat16) for i in range(BB)], axis=1)
    x_prev = jnp.where(m_first, zero, pltpu.roll(xs, shift=1, axis=1))
    x_next = jnp.where(m_last, zero, pltpu.roll(xs, shift=L - 1, axis=1))

    y = jnp.dot(w_ref[0], x_prev, preferred_element_type=jnp.float32)
    y = y + jnp.dot(w_ref[1], xs, preferred_element_type=jnp.float32)
    y = y + jnp.dot(w_ref[2], x_next, preferred_element_type=jnp.float32)
    y = y + b_ref[...]

    for i in range(BB):
        o_ref[i] = y[:, i * T:(i + 1) * T]


def kernel(mel_spec, w_taps, bias_col):
    B, M, T = mel_spec.shape
    BB = 8
    w_folded = w_taps.at[1].add(jnp.float32(0.7) * jnp.eye(M, dtype=w_taps.dtype))
    w_bf16 = w_folded.astype(jnp.bfloat16)

    return pl.pallas_call(
        functools.partial(_enhancer_kernel, T=T, BB=BB),
        out_shape=jax.ShapeDtypeStruct((B, M, T), mel_spec.dtype),
        grid=(B // BB,),
        in_specs=[
            pl.BlockSpec((3, M, M), lambda b: (0, 0, 0)),
            pl.BlockSpec((M, 1), lambda b: (0, 0)),
            pl.BlockSpec((BB, M, T), lambda b: (b, 0, 0)),
        ],
        out_specs=pl.BlockSpec((BB, M, T), lambda b: (b, 0, 0)),
        compiler_params=pltpu.CompilerParams(
            dimension_semantics=("parallel",),
            vmem_limit_bytes=64 << 20,
        ),
    )(w_bf16, bias_col, mel_spec)
